# baseline (speedup 1.0000x reference)
import functools
import math

import jax
import jax.numpy as jnp
from jax import lax
from jax.experimental import pallas as pl
from jax.experimental.pallas import tpu as pltpu
from jax.experimental.pallas import tpu_sc as plsc

EMB = 64
LANES = 16
NW = 32


def _make_sc_embed(bsz: int, seq: int, scale: float):
    mesh = plsc.VectorSubcoreMesh(core_axis_name="c", subcore_axis_name="s")
    rows_per_w = bsz // NW

    scratch = [pltpu.VMEM((rows_per_w, seq), jnp.int32)]
    scratch += [pltpu.VMEM((seq, EMB), jnp.float32) for _ in range(4)]
    scratch += [pltpu.SemaphoreType.DMA for _ in range(4)]

    @functools.partial(
        pl.kernel,
        mesh=mesh,
        out_type=jax.ShapeDtypeStruct((bsz, seq, EMB), jnp.float32),
        scratch_types=scratch,
        compiler_params=pltpu.CompilerParams(use_tc_tiling_on_sc=False),
    )
    def sc_embed(tokens_hbm, table_hbm, out_hbm,
                 idx_v, in0, in1, ob0, ob1, g0, g1, s0, s1):
        ins = (in0, in1)
        obs = (ob0, ob1)
        gsem = (g0, g1)
        ssem = (s0, s1)

        nc = lax.axis_size("c")
        wid = lax.axis_index("s") * nc + lax.axis_index("c")
        row0 = wid * rows_per_w
        pltpu.sync_copy(tokens_hbm.at[pl.ds(row0, rows_per_w)], idx_v)

        def fire_gather(t, b):
            pltpu.async_copy(table_hbm.at[idx_v.at[t]], ins[b], gsem[b])

        def wait_gather(t, b):
            pltpu.make_async_copy(
                table_hbm.at[idx_v.at[t]], ins[b], gsem[b]
            ).wait()

        def fire_store(t, b):
            pltpu.async_copy(obs[b], out_hbm.at[row0 + t], ssem[b])

        def wait_store(b):
            pltpu.make_async_copy(obs[b], out_hbm.at[row0], ssem[b]).wait()

        def scale_chunk(b):
            def scale_row(r, c2):
                for v in range(EMB // LANES):
                    sl = pl.ds(v * LANES, LANES)
                    obs[b][r, sl] = ins[b][r, sl] * scale
                return c2

            lax.fori_loop(0, seq, scale_row, 0, unroll=10)

        fire_gather(0, 0)
        fire_gather(1, 1)
        for t in range(2):
            wait_gather(t, t)
            scale_chunk(t)
            fire_store(t, t)
            fire_gather(t + 2, t)

        def pair(p, carry):
            for b in range(2):
                t = 2 + p * 2 + b
                wait_store(b)
                wait_gather(t, b)
                scale_chunk(b)
                fire_store(t, b)
                fire_gather(t + 2, b)
            return carry

        lax.fori_loop(0, (rows_per_w - 4) // 2, pair, 0)

        for tt in range(2):
            t = rows_per_w - 2 + tt
            b = t % 2
            wait_store(b)
            wait_gather(t, b)
            scale_chunk(b)
            fire_store(t, b)

        wait_store(0)
        wait_store(1)

    return sc_embed


def kernel(tokens, table):
    bsz, seq = tokens.shape
    assert bsz % NW == 0 and (bsz // NW) % 2 == 0
    scale = math.sqrt(float(EMB))
    return _make_sc_embed(bsz, seq, scale)(tokens.astype(jnp.int32), table)


# speedup vs baseline: 1.0037x; 1.0037x over previous
import functools
import math

import jax
import jax.numpy as jnp
from jax import lax
from jax.experimental import pallas as pl
from jax.experimental.pallas import tpu as pltpu
from jax.experimental.pallas import tpu_sc as plsc

EMB = 64
LANES = 16
NW = 32
ROWS_PER_CHUNK = 4


def _make_sc_embed(bsz: int, seq: int, scale: float):
    mesh = plsc.VectorSubcoreMesh(core_axis_name="c", subcore_axis_name="s")
    chunk = ROWS_PER_CHUNK * seq
    nchunk = bsz // (NW * ROWS_PER_CHUNK)
    ngroups = -(-chunk // LANES)
    tokpad = ngroups * LANES
    split = (chunk // 2 + 7) & ~7

    scratch = []
    scratch += [pltpu.VMEM((tokpad,), jnp.int32) for _ in range(2)]
    scratch += [pltpu.VMEM((tokpad,), jnp.int32) for _ in range(2)]
    scratch += [pltpu.VMEM((chunk, 2 * EMB), jnp.float32) for _ in range(2)]
    scratch += [pltpu.VMEM((ROWS_PER_CHUNK, seq, EMB), jnp.float32)
                for _ in range(2)]
    scratch += [pltpu.SemaphoreType.DMA for _ in range(6)]

    @functools.partial(
        pl.kernel,
        mesh=mesh,
        out_type=jax.ShapeDtypeStruct((bsz, seq, EMB), jnp.float32),
        scratch_types=scratch,
        compiler_params=pltpu.CompilerParams(use_tc_tiling_on_sc=True),
    )
    def sc_embed(tokens_hbm, table_hbm, out_hbm,
                 tk0, tk1, px0, px1, in0, in1, ob0, ob1,
                 t0, t1, g0, g1, s0, s1):
        tks = (tk0, tk1)
        pxs = (px0, px1)
        ins = (in0, in1)
        obs = (ob0, ob1)
        tsem = (t0, t1)
        gsem = (g0, g1)
        ssem = (s0, s1)

        nc = lax.axis_size("c")
        wid = lax.axis_index("s") * nc + lax.axis_index("c")
        tok0 = wid * nchunk * chunk
        row0 = wid * nchunk * ROWS_PER_CHUNK

        def fire_tok(t, b):
            pltpu.async_copy(
                tokens_hbm.at[pl.ds(tok0 + t * chunk, chunk)],
                tks[b].at[pl.ds(0, chunk)], tsem[b])

        def wait_tok(b):
            pltpu.make_async_copy(
                tokens_hbm.at[pl.ds(tok0, chunk)],
                tks[b].at[pl.ds(0, chunk)], tsem[b]).wait()

        def prep_idx(b):
            for g in range(ngroups):
                sl = pl.ds(g * LANES, LANES)
                pxs[b][sl] = tks[b][sl] >> 1

        def fire_gather(b):
            pltpu.async_copy(
                table_hbm.at[pxs[b].at[pl.ds(0, split)]],
                ins[b].at[pl.ds(0, split)], gsem[b])
            pltpu.async_copy(
                table_hbm.at[pxs[b].at[pl.ds(split, chunk - split)]],
                ins[b].at[pl.ds(split, chunk - split)], gsem[b])

        def wait_gather(b):
            pltpu.make_async_copy(
                table_hbm.at[pxs[b].at[pl.ds(0, split)]],
                ins[b].at[pl.ds(0, split)], gsem[b]).wait()
            pltpu.make_async_copy(
                table_hbm.at[pxs[b].at[pl.ds(split, chunk - split)]],
                ins[b].at[pl.ds(split, chunk - split)], gsem[b]).wait()

        def fire_store(t, b):
            pltpu.async_copy(
                obs[b], out_hbm.at[pl.ds(row0 + t * ROWS_PER_CHUNK,
                                         ROWS_PER_CHUNK)], ssem[b])

        def wait_store(b):
            pltpu.make_async_copy(
                obs[b], out_hbm.at[pl.ds(row0, ROWS_PER_CHUNK)],
                ssem[b]).wait()

        def scale_chunk(b):
            for g in range(ngroups):
                hvec = (tks[b][pl.ds(g * LANES, LANES)] & 1) * EMB
                for l in range(LANES):
                    tkn = g * LANES + l
                    if tkn >= chunk:
                        break
                    off = jnp.squeeze(lax.slice(hvec, (l,), (l + 1,)))
                    p, r = divmod(tkn, seq)
                    for v in range(EMB // LANES):
                        obs[b][p, r, pl.ds(v * LANES, LANES)] = (
                            ins[b][tkn, pl.ds(off + v * LANES, LANES)] * scale)

        fire_tok(0, 0)
        fire_tok(1, 1)
        wait_tok(0)
        prep_idx(0)
        fire_gather(0)

        def body_static(t, b):
            @pl.when(t + 1 < nchunk)
            def _prep_next():
                wait_tok(1 - b)
                prep_idx(1 - b)
                fire_gather(1 - b)

            @pl.when(t + 2 < nchunk)
            def _stage_next():
                fire_tok(t + 2, b)

            @pl.when(t >= 2)
            def _drain_store():
                wait_store(b)

            wait_gather(b)
            scale_chunk(b)
            fire_store(t, b)
            return 0

        def pair_body(p, carry):
            body_static(2 * p, 0)
            body_static(2 * p + 1, 1)
            return carry

        lax.fori_loop(0, nchunk // 2, pair_body, 0)

        wait_store(0)
        wait_store(1)

    return sc_embed


def kernel(tokens, table):
    bsz, seq = tokens.shape
    vocab = table.shape[0]
    assert bsz % (NW * ROWS_PER_CHUNK) == 0 and vocab % 2 == 0
    scale = math.sqrt(float(EMB))
    pair_table = table.reshape(vocab // 2, 2 * EMB)
    flat_tokens = tokens.reshape(-1).astype(jnp.int32)
    return _make_sc_embed(bsz, seq, scale)(flat_tokens, pair_table)
